# baseline (speedup 1.0000x reference)
import functools

import jax
import jax.numpy as jnp
from jax import lax
from jax.experimental import pallas as pl
from jax.experimental.pallas import tpu as pltpu
from jax.experimental.pallas import tpu_sc as plsc

_B, _C, _H, _W = 4, 384, 224, 224
_HW = _H * _W
_K = 8192
_L = 16

_HBLK = 16


def _loss_body(x_ref, y_ref, o_ref):
    d = x_ref[0] - y_ref[0]
    o_ref[0] = jnp.sum(d * d, axis=-1) * jnp.float32(1.0 / _C)


def _loss_map(x, y, phase):
    xt = x.transpose(0, 2, 3, 1)
    yt = y.transpose(0, 2, 3, 1)
    grid = (2, _H // _HBLK)
    in_spec = pl.BlockSpec((1, _HBLK, _W, _C),
                           lambda b, h: (2 * phase + b, h, 0, 0))
    out_spec = pl.BlockSpec((1, _HBLK, _W), lambda b, h: (b, h, 0))
    out = pl.pallas_call(
        _loss_body,
        grid=grid,
        in_specs=[in_spec, in_spec],
        out_specs=out_spec,
        out_shape=jax.ShapeDtypeStruct((2, _H, _W), jnp.float32),
        compiler_params=pltpu.CompilerParams(
            dimension_semantics=("parallel", "parallel"),
        ),
    )(xt, yt)
    return out.reshape(2, _HW)


_SHIFTS = (23, 15, 7, 0)
_MASKS = (0xFF, 0xFF, 0xFF, 0x7F)


_TPB = 16
_SLICE = 3200
_SLICE_LAST = _HW - 15 * _SLICE


def _radix_topk_body(loss_hbm, out_hbm, buf, hcnt, hsum, lcnt, lsum,
                     rcnt, rsum, outv, shc, shs, b, t):
    lane = lax.iota(jnp.int32, _L)

    @pl.when(t < _TPB - 1)
    def _():
        pltpu.sync_copy(loss_hbm.at[b, 0, pl.ds(t * _SLICE, _SLICE)],
                        buf.at[pl.ds(0, _SLICE)])

    @pl.when(t == _TPB - 1)
    def _():
        pltpu.sync_copy(loss_hbm.at[b, 0, pl.ds(15 * _SLICE, _SLICE_LAST)],
                        buf.at[pl.ds(0, _SLICE_LAST)])

    n = jnp.where(t == _TPB - 1, jnp.int32(_SLICE_LAST), jnp.int32(_SLICE))
    need = jnp.int32(_K)
    acc_cnt = jnp.int32(0)
    acc_sum = jnp.float32(0.0)
    thr_bits = jnp.int32(0)

    for p in range(4):
        sh, mk = _SHIFTS[p], _MASKS[p]
        nbin = mk + 1
        nchunk = nbin // _L

        def zero_body(j, _):
            hcnt[pl.ds(j * _L, _L)] = jnp.zeros((_L,), jnp.int32)
            hsum[pl.ds(j * _L, _L)] = jnp.zeros((_L,), jnp.float32)
            return 0

        lax.fori_loop(0, nbin, zero_body, 0)

        nvr = (n + _L - 1) // _L
        ones = jnp.ones((_L,), jnp.int32)

        def hist_body(i, _, sh=sh, mk=mk, n=n):
            v = buf[pl.ds(i * _L, _L)]
            bits = lax.bitcast_convert_type(v, jnp.int32)
            valid = (i * _L + lane) < n
            binv = (bits >> sh) & mk
            addr = lane * 256 + binv
            plsc.addupdate_scatter(hcnt, [addr], ones, mask=valid)
            plsc.addupdate_scatter(hsum, [addr], v, mask=valid)
            return 0

        lax.fori_loop(0, nvr, hist_body, 0)

        for j in range(nchunk):
            def fold_body(l, carry, j=j):
                ca, sa = carry
                ca = ca + hcnt[pl.ds(l * 256 + j * _L, _L)]
                sa = sa + hsum[pl.ds(l * 256 + j * _L, _L)]
                return ca, sa

            ca, sa = lax.fori_loop(
                0, _L, fold_body,
                (jnp.zeros((_L,), jnp.int32), jnp.zeros((_L,), jnp.float32)))
            lcnt[pl.ds(j * _L, _L)] = ca
            lsum[pl.ds(j * _L, _L)] = sa

        pltpu.sync_copy(lcnt, shc.at[p, pl.ds(t * 256, 256)])
        pltpu.sync_copy(lsum, shs.at[p, pl.ds(t * 256, 256)])
        plsc.subcore_barrier()
        pltpu.sync_copy(shc.at[p], rcnt)
        pltpu.sync_copy(shs.at[p], rsum)

        cgs, sgs = [], []
        for j in range(nchunk):
            cg = rcnt[pl.ds(j * _L, _L)]
            sg = rsum[pl.ds(j * _L, _L)]
            for t2 in range(1, _TPB):
                cg = cg + rcnt[pl.ds(t2 * 256 + j * _L, _L)]
                sg = sg + rsum[pl.ds(t2 * 256 + j * _L, _L)]
            cgs.append(cg)
            sgs.append(sg)

        tc = [jnp.sum(cg) for cg in cgs]
        ts = [jnp.sum(sg) for sg in sgs]
        sbc = [jnp.int32(0)] * nchunk
        sbs = [jnp.float32(0.0)] * nchunk
        for j in range(nchunk - 2, -1, -1):
            sbc[j] = sbc[j + 1] + tc[j + 1]
            sbs[j] = sbs[j + 1] + ts[j + 1]

        beta_cnt = jnp.int32(0)
        delta_cnt = jnp.int32(0)
        delta_sum = jnp.float32(0.0)
        for j in range(nchunk):
            cnt_ge = sbc[j] + (tc[j] - plsc.cumsum(cgs[j]) + cgs[j])
            sum_ge = sbs[j] + (ts[j] - plsc.cumsum(sgs[j]) + sgs[j])
            ind = cnt_ge >= need
            beta_cnt = beta_cnt + jnp.sum(jnp.where(ind, 1, 0))
            lt = jnp.logical_not(ind)
            delta_cnt = jnp.maximum(delta_cnt, jnp.max(jnp.where(lt, cnt_ge, 0)))
            delta_sum = jnp.maximum(
                delta_sum, jnp.max(jnp.where(lt, sum_ge, jnp.float32(0.0))))
        beta = beta_cnt - 1

        acc_cnt = acc_cnt + delta_cnt
        acc_sum = acc_sum + delta_sum
        need = need - delta_cnt
        thr_bits = thr_bits | (beta << sh)

        if p < 3:
            def comp_body(i, off, sh=sh, mk=mk, beta=beta, n=n):
                v = buf[pl.ds(i * _L, _L)]
                bits = lax.bitcast_convert_type(v, jnp.int32)
                valid = (i * _L + lane) < n
                m = jnp.logical_and(valid, ((bits >> sh) & mk) == beta)
                plsc.store_compressed(buf.at[pl.ds(off, _L)], v, mask=m)
                return off + jnp.sum(jnp.where(m, 1, 0))

            n = lax.fori_loop(0, nvr, comp_body, jnp.int32(0))

    @pl.when(t == 0)
    def _():
        thr_vec = lax.bitcast_convert_type(
            jnp.full((_L,), thr_bits, jnp.int32), jnp.float32)
        thr = jnp.max(thr_vec)
        total = acc_sum + thr * (need).astype(jnp.float32)
        outv[...] = jnp.full((_L,), total * jnp.float32(1.0 / (_B * _K)))
        pltpu.sync_copy(outv, out_hbm.at[b, 0])


def _topk_mean(loss):
    mesh = plsc.VectorSubcoreMesh(core_axis_name="c", subcore_axis_name="s")

    @functools.partial(
        pl.kernel,
        out_type=jax.ShapeDtypeStruct((2, 1, _L), jnp.float32),
        mesh=mesh,
        compiler_params=pltpu.CompilerParams(needs_layout_passes=False),
        scratch_types=[
            pltpu.VMEM((_SLICE + _L,), jnp.float32),
            pltpu.VMEM((256 * _L,), jnp.int32),
            pltpu.VMEM((256 * _L,), jnp.float32),
            pltpu.VMEM((256,), jnp.int32),
            pltpu.VMEM((256,), jnp.float32),
            pltpu.VMEM((_TPB * 256,), jnp.int32),
            pltpu.VMEM((_TPB * 256,), jnp.float32),
            pltpu.VMEM((_L,), jnp.float32),
            pltpu.VMEM_SHARED((4, _TPB * 256), jnp.int32),
            pltpu.VMEM_SHARED((4, _TPB * 256), jnp.float32),
        ],
    )
    def k(loss_hbm, out_hbm, buf, hcnt, hsum, lcnt, lsum, rcnt, rsum,
          outv, shc, shs):
        cid = lax.axis_index("c")
        sid = lax.axis_index("s")
        _radix_topk_body(loss_hbm, out_hbm, buf, hcnt, hsum, lcnt, lsum,
                         rcnt, rsum, outv, shc, shs, cid, sid)

    return k(loss.reshape(2, 1, _HW))


def kernel(x, y):
    loss_a = _loss_map(x, y, 0)
    parts_a = _topk_mean(loss_a)
    loss_b = _loss_map(x, y, 1)
    parts_b = _topk_mean(loss_b)
    return jnp.sum(parts_a[:, 0, 0]) + jnp.sum(parts_b[:, 0, 0])


# speedup vs baseline: 1.0548x; 1.0548x over previous
import functools

import jax
import jax.numpy as jnp
from jax import lax
from jax.experimental import pallas as pl
from jax.experimental.pallas import tpu as pltpu
from jax.experimental.pallas import tpu_sc as plsc

_B, _C, _H, _W = 4, 384, 224, 224
_HW = _H * _W
_K = 8192
_L = 16

_HBLK = 16


def _loss_body(x_ref, y_ref, o_ref):
    d = x_ref[0] - y_ref[0]
    o_ref[0] = jnp.sum(d * d, axis=-1) * jnp.float32(1.0 / _C)


def _loss_map(x, y):
    xt = x.transpose(0, 2, 3, 1)
    yt = y.transpose(0, 2, 3, 1)
    grid = (_B, _H // _HBLK)
    in_spec = pl.BlockSpec((1, _HBLK, _W, _C), lambda b, h: (b, h, 0, 0))
    out_spec = pl.BlockSpec((1, _HBLK, _W), lambda b, h: (b, h, 0))
    out = pl.pallas_call(
        _loss_body,
        grid=grid,
        in_specs=[in_spec, in_spec],
        out_specs=out_spec,
        out_shape=jax.ShapeDtypeStruct((_B, _H, _W), jnp.float32),
        compiler_params=pltpu.CompilerParams(
            dimension_semantics=("parallel", "parallel"),
        ),
    )(xt, yt)
    return out.reshape(_B, _HW)


_SHIFTS = (23, 15, 7, 0)
_MASKS = (0xFF, 0xFF, 0xFF, 0x7F)


_TPB = 8
_SLICE = _HW // _TPB


def _radix_topk_body(loss_hbm, out_hbm, buf, hcnt, hsum, lcnt, lsum,
                     rcnt, rsum, outv, shc, shs, b, lb, t):
    lane = lax.iota(jnp.int32, _L)
    pltpu.sync_copy(loss_hbm.at[b, pl.ds(t * _SLICE, _SLICE)],
                    buf.at[pl.ds(0, _SLICE)])

    n = jnp.int32(_SLICE)
    need = jnp.int32(_K)
    acc_cnt = jnp.int32(0)
    acc_sum = jnp.float32(0.0)
    thr_bits = jnp.int32(0)

    for p in range(4):
        sh, mk = _SHIFTS[p], _MASKS[p]
        nbin = mk + 1
        nchunk = nbin // _L

        @plsc.parallel_loop(0, nbin, unroll=4)
        def _(j):
            hcnt[pl.ds(j * _L, _L)] = jnp.zeros((_L,), jnp.int32)
            hsum[pl.ds(j * _L, _L)] = jnp.zeros((_L,), jnp.float32)

        nvr = (n + _L - 1) // _L
        ones = jnp.ones((_L,), jnp.int32)

        def hist_body(i, sh=sh, mk=mk, n=n):
            v = buf[pl.ds(i * _L, _L)]
            bits = lax.bitcast_convert_type(v, jnp.int32)
            valid = (i * _L + lane) < n
            binv = (bits >> sh) & mk
            addr = lane * 256 + binv
            plsc.addupdate_scatter(hcnt, [addr], ones, mask=valid)
            plsc.addupdate_scatter(hsum, [addr], v, mask=valid)

        plsc.parallel_loop(0, nvr, unroll=4)(hist_body)

        for j in range(nchunk):
            def fold_body(l, carry, j=j):
                ca, sa = carry
                ca = ca + hcnt[pl.ds(l * 256 + j * _L, _L)]
                sa = sa + hsum[pl.ds(l * 256 + j * _L, _L)]
                return ca, sa

            ca, sa = lax.fori_loop(
                0, _L, fold_body,
                (jnp.zeros((_L,), jnp.int32), jnp.zeros((_L,), jnp.float32)))
            lcnt[pl.ds(j * _L, _L)] = ca
            lsum[pl.ds(j * _L, _L)] = sa

        pltpu.sync_copy(lcnt, shc.at[p, lb, pl.ds(t * 256, 256)])
        pltpu.sync_copy(lsum, shs.at[p, lb, pl.ds(t * 256, 256)])
        plsc.subcore_barrier()
        pltpu.sync_copy(shc.at[p, lb], rcnt)
        pltpu.sync_copy(shs.at[p, lb], rsum)

        cgs, sgs = [], []
        for j in range(nchunk):
            cg = rcnt[pl.ds(j * _L, _L)]
            sg = rsum[pl.ds(j * _L, _L)]
            for t2 in range(1, _TPB):
                cg = cg + rcnt[pl.ds(t2 * 256 + j * _L, _L)]
                sg = sg + rsum[pl.ds(t2 * 256 + j * _L, _L)]
            cgs.append(cg)
            sgs.append(sg)

        tc = [jnp.sum(cg) for cg in cgs]
        ts = [jnp.sum(sg) for sg in sgs]
        sbc = [jnp.int32(0)] * nchunk
        sbs = [jnp.float32(0.0)] * nchunk
        for j in range(nchunk - 2, -1, -1):
            sbc[j] = sbc[j + 1] + tc[j + 1]
            sbs[j] = sbs[j + 1] + ts[j + 1]

        beta_cnt = jnp.int32(0)
        delta_cnt = jnp.int32(0)
        delta_sum = jnp.float32(0.0)
        for j in range(nchunk):
            cnt_ge = sbc[j] + (tc[j] - plsc.cumsum(cgs[j]) + cgs[j])
            sum_ge = sbs[j] + (ts[j] - plsc.cumsum(sgs[j]) + sgs[j])
            ind = cnt_ge >= need
            beta_cnt = beta_cnt + jnp.sum(jnp.where(ind, 1, 0))
            lt = jnp.logical_not(ind)
            delta_cnt = jnp.maximum(delta_cnt, jnp.max(jnp.where(lt, cnt_ge, 0)))
            delta_sum = jnp.maximum(
                delta_sum, jnp.max(jnp.where(lt, sum_ge, jnp.float32(0.0))))
        beta = beta_cnt - 1

        acc_cnt = acc_cnt + delta_cnt
        acc_sum = acc_sum + delta_sum
        need = need - delta_cnt
        thr_bits = thr_bits | (beta << sh)

        if p < 3:
            def comp_body(i, off, sh=sh, mk=mk, beta=beta, n=n):
                v = buf[pl.ds(i * _L, _L)]
                bits = lax.bitcast_convert_type(v, jnp.int32)
                valid = (i * _L + lane) < n
                m = jnp.logical_and(valid, ((bits >> sh) & mk) == beta)
                plsc.store_compressed(buf.at[pl.ds(off, _L)], v, mask=m)
                return off + jnp.sum(jnp.where(m, 1, 0))

            n = lax.fori_loop(0, nvr, comp_body, jnp.int32(0))

    @pl.when(t == 0)
    def _():
        thr_vec = lax.bitcast_convert_type(
            jnp.full((_L,), thr_bits, jnp.int32), jnp.float32)
        thr = jnp.max(thr_vec)
        total = acc_sum + thr * (need).astype(jnp.float32)
        outv[...] = jnp.full((_L,), total * jnp.float32(1.0 / (_B * _K)))
        pltpu.sync_copy(outv, out_hbm.at[b])


def _topk_mean(loss):
    mesh = plsc.VectorSubcoreMesh(core_axis_name="c", subcore_axis_name="s")

    @functools.partial(
        pl.kernel,
        out_type=jax.ShapeDtypeStruct((_B, _L), jnp.float32),
        mesh=mesh,
        compiler_params=pltpu.CompilerParams(needs_layout_passes=False),
        scratch_types=[
            pltpu.VMEM((_SLICE + _L,), jnp.float32),
            pltpu.VMEM((256 * _L,), jnp.int32),
            pltpu.VMEM((256 * _L,), jnp.float32),
            pltpu.VMEM((256,), jnp.int32),
            pltpu.VMEM((256,), jnp.float32),
            pltpu.VMEM((_TPB * 256,), jnp.int32),
            pltpu.VMEM((_TPB * 256,), jnp.float32),
            pltpu.VMEM((_L,), jnp.float32),
            pltpu.VMEM_SHARED((4, 2, _TPB * 256), jnp.int32),
            pltpu.VMEM_SHARED((4, 2, _TPB * 256), jnp.float32),
        ],
    )
    def k(loss_hbm, out_hbm, buf, hcnt, hsum, lcnt, lsum, rcnt, rsum,
          outv, shc, shs):
        cid = lax.axis_index("c")
        sid = lax.axis_index("s")
        lb = sid // _TPB
        t = sid % _TPB
        b = cid * 2 + lb
        _radix_topk_body(loss_hbm, out_hbm, buf, hcnt, hsum, lcnt, lsum,
                         rcnt, rsum, outv, shc, shs, b, lb, t)

    return k(loss)


def kernel(x, y):
    loss = _loss_map(x, y)
    parts = _topk_mean(loss)
    return jnp.sum(parts[:, 0])


# speedup vs baseline: 1.0753x; 1.0195x over previous
import functools

import jax
import jax.numpy as jnp
from jax import lax
from jax.experimental import pallas as pl
from jax.experimental.pallas import tpu as pltpu
from jax.experimental.pallas import tpu_sc as plsc

_B, _C, _H, _W = 4, 384, 224, 224
_HW = _H * _W
_K = 8192
_L = 16

_HBLK = 16


def _loss_body(x_ref, y_ref, o_ref):
    d = x_ref[0] - y_ref[0]
    o_ref[0] = jnp.sum(d * d, axis=-1) * jnp.float32(1.0 / _C)


def _loss_map(x, y):
    xt = x.transpose(0, 2, 3, 1)
    yt = y.transpose(0, 2, 3, 1)
    grid = (_B, _H // _HBLK)
    in_spec = pl.BlockSpec((1, _HBLK, _W, _C), lambda b, h: (b, h, 0, 0))
    out_spec = pl.BlockSpec((1, _HBLK, _W), lambda b, h: (b, h, 0))
    out = pl.pallas_call(
        _loss_body,
        grid=grid,
        in_specs=[in_spec, in_spec],
        out_specs=out_spec,
        out_shape=jax.ShapeDtypeStruct((_B, _H, _W), jnp.float32),
        compiler_params=pltpu.CompilerParams(
            dimension_semantics=("parallel", "parallel"),
        ),
    )(xt, yt)
    return out.reshape(_B, _HW)


_SHIFTS = (23, 15, 7, 0)
_MASKS = (0xFF, 0xFF, 0xFF, 0x7F)


_TPB = 8
_SLICE = _HW // _TPB


def _radix_topk_body(loss_hbm, out_hbm, buf, buf2, hcnt, hsum, lcnt, lsum,
                     rcnt, rsum, outv, shc, shs, b, lb, t):
    lane = lax.iota(jnp.int32, _L)
    pltpu.sync_copy(loss_hbm.at[b, pl.ds(t * _SLICE, _SLICE)],
                    buf.at[pl.ds(0, _SLICE)])

    n = jnp.int32(_SLICE)
    need = jnp.int32(_K)
    acc_cnt = jnp.int32(0)
    acc_sum = jnp.float32(0.0)
    thr_bits = jnp.int32(0)

    for p in range(4):
        sh, mk = _SHIFTS[p], _MASKS[p]
        src_buf = buf if p % 2 == 0 else buf2
        dst_buf = buf2 if p % 2 == 0 else buf
        nbin = mk + 1
        nchunk = nbin // _L

        @plsc.parallel_loop(0, nbin, unroll=4)
        def _(j):
            hcnt[pl.ds(j * _L, _L)] = jnp.zeros((_L,), jnp.int32)
            hsum[pl.ds(j * _L, _L)] = jnp.zeros((_L,), jnp.float32)

        nvr = (n + _L - 1) // _L
        ones = jnp.ones((_L,), jnp.int32)

        def hist_body(i, sh=sh, mk=mk, n=n, src_buf=src_buf):
            v = src_buf[pl.ds(i * _L, _L)]
            bits = lax.bitcast_convert_type(v, jnp.int32)
            valid = (i * _L + lane) < n
            binv = (bits >> sh) & mk
            addr = lane * 256 + binv
            plsc.addupdate_scatter(hcnt, [addr], ones, mask=valid)
            plsc.addupdate_scatter(hsum, [addr], v, mask=valid)

        plsc.parallel_loop(0, nvr, unroll=4)(hist_body)

        for j in range(nchunk):
            def fold_body(l, carry, j=j):
                ca, sa = carry
                ca = ca + hcnt[pl.ds(l * 256 + j * _L, _L)]
                sa = sa + hsum[pl.ds(l * 256 + j * _L, _L)]
                return ca, sa

            ca, sa = lax.fori_loop(
                0, _L, fold_body,
                (jnp.zeros((_L,), jnp.int32), jnp.zeros((_L,), jnp.float32)))
            lcnt[pl.ds(j * _L, _L)] = ca
            lsum[pl.ds(j * _L, _L)] = sa

        pltpu.sync_copy(lcnt, shc.at[p, lb, pl.ds(t * 256, 256)])
        pltpu.sync_copy(lsum, shs.at[p, lb, pl.ds(t * 256, 256)])
        plsc.subcore_barrier()
        pltpu.sync_copy(shc.at[p, lb], rcnt)
        pltpu.sync_copy(shs.at[p, lb], rsum)

        cgs, sgs = [], []
        for j in range(nchunk):
            cg = rcnt[pl.ds(j * _L, _L)]
            sg = rsum[pl.ds(j * _L, _L)]
            for t2 in range(1, _TPB):
                cg = cg + rcnt[pl.ds(t2 * 256 + j * _L, _L)]
                sg = sg + rsum[pl.ds(t2 * 256 + j * _L, _L)]
            cgs.append(cg)
            sgs.append(sg)

        tc = [jnp.sum(cg) for cg in cgs]
        ts = [jnp.sum(sg) for sg in sgs]
        sbc = [jnp.int32(0)] * nchunk
        sbs = [jnp.float32(0.0)] * nchunk
        for j in range(nchunk - 2, -1, -1):
            sbc[j] = sbc[j + 1] + tc[j + 1]
            sbs[j] = sbs[j + 1] + ts[j + 1]

        beta_cnt = jnp.int32(0)
        delta_cnt = jnp.int32(0)
        delta_sum = jnp.float32(0.0)
        for j in range(nchunk):
            cnt_ge = sbc[j] + (tc[j] - plsc.cumsum(cgs[j]) + cgs[j])
            sum_ge = sbs[j] + (ts[j] - plsc.cumsum(sgs[j]) + sgs[j])
            ind = cnt_ge >= need
            beta_cnt = beta_cnt + jnp.sum(jnp.where(ind, 1, 0))
            lt = jnp.logical_not(ind)
            delta_cnt = jnp.maximum(delta_cnt, jnp.max(jnp.where(lt, cnt_ge, 0)))
            delta_sum = jnp.maximum(
                delta_sum, jnp.max(jnp.where(lt, sum_ge, jnp.float32(0.0))))
        beta = beta_cnt - 1

        acc_cnt = acc_cnt + delta_cnt
        acc_sum = acc_sum + delta_sum
        need = need - delta_cnt
        thr_bits = thr_bits | (beta << sh)

        if p < 3:
            def comp_body(i, off, sh=sh, mk=mk, beta=beta, n=n,
                          src_buf=src_buf, dst_buf=dst_buf):
                v = src_buf[pl.ds(i * _L, _L)]
                bits = lax.bitcast_convert_type(v, jnp.int32)
                valid = (i * _L + lane) < n
                m = jnp.logical_and(valid, ((bits >> sh) & mk) == beta)
                plsc.store_compressed(dst_buf.at[pl.ds(off, _L)], v, mask=m)
                return off + jnp.sum(jnp.where(m, 1, 0))

            n = plsc.parallel_loop(0, nvr, unroll=4,
                                   carry=jnp.int32(0))(comp_body)

    @pl.when(t == 0)
    def _():
        thr_vec = lax.bitcast_convert_type(
            jnp.full((_L,), thr_bits, jnp.int32), jnp.float32)
        thr = jnp.max(thr_vec)
        total = acc_sum + thr * (need).astype(jnp.float32)
        outv[...] = jnp.full((_L,), total * jnp.float32(1.0 / (_B * _K)))
        pltpu.sync_copy(outv, out_hbm.at[b])


def _topk_mean(loss):
    mesh = plsc.VectorSubcoreMesh(core_axis_name="c", subcore_axis_name="s")

    @functools.partial(
        pl.kernel,
        out_type=jax.ShapeDtypeStruct((_B, _L), jnp.float32),
        mesh=mesh,
        compiler_params=pltpu.CompilerParams(needs_layout_passes=False),
        scratch_types=[
            pltpu.VMEM((_SLICE + _L,), jnp.float32),
            pltpu.VMEM((_SLICE + _L,), jnp.float32),
            pltpu.VMEM((256 * _L,), jnp.int32),
            pltpu.VMEM((256 * _L,), jnp.float32),
            pltpu.VMEM((256,), jnp.int32),
            pltpu.VMEM((256,), jnp.float32),
            pltpu.VMEM((_TPB * 256,), jnp.int32),
            pltpu.VMEM((_TPB * 256,), jnp.float32),
            pltpu.VMEM((_L,), jnp.float32),
            pltpu.VMEM_SHARED((4, 2, _TPB * 256), jnp.int32),
            pltpu.VMEM_SHARED((4, 2, _TPB * 256), jnp.float32),
        ],
    )
    def k(loss_hbm, out_hbm, buf, buf2, hcnt, hsum, lcnt, lsum, rcnt, rsum,
          outv, shc, shs):
        cid = lax.axis_index("c")
        sid = lax.axis_index("s")
        lb = sid // _TPB
        t = sid % _TPB
        b = cid * 2 + lb
        _radix_topk_body(loss_hbm, out_hbm, buf, buf2, hcnt, hsum, lcnt, lsum,
                         rcnt, rsum, outv, shc, shs, b, lb, t)

    return k(loss)


def kernel(x, y):
    loss = _loss_map(x, y)
    parts = _topk_mean(loss)
    return jnp.sum(parts[:, 0])
